# baseline (speedup 1.0000x reference)
import functools

import jax
import jax.numpy as jnp
from jax import lax
from jax.experimental import pallas as pl
from jax.experimental.pallas import tpu as pltpu
from jax.experimental.pallas import tpu_sc as plsc

_NC = 2
_NS = 16
_NW = _NC * _NS
_CHUNK = 128
_L = 16


def kernel(user_ids, table):
    B = user_ids.shape[0]
    V, D = table.shape
    rpsr = 128 // D
    b_per_w = B // _NW
    n_chunks = b_per_w // _CHUNK
    n_groups = _CHUNK // _L

    table128 = table.reshape(V // rpsr, D * rpsr)
    ids = user_ids.astype(jnp.int32)
    super_idx = (ids // rpsr).reshape(_NW * n_chunks, _CHUNK)
    sub_off = ((ids % rpsr) * D).reshape(_NW, b_per_w)

    mesh = plsc.VectorSubcoreMesh(core_axis_name="c", subcore_axis_name="s")

    @functools.partial(
        pl.kernel,
        out_type=jax.ShapeDtypeStruct((B, D), jnp.float32),
        mesh=mesh,
        scratch_types=[
            pltpu.VMEM((n_chunks, _CHUNK), jnp.int32),
            pltpu.VMEM((b_per_w,), jnp.int32),
            pltpu.VMEM((2, _CHUNK, D * rpsr), jnp.float32),
            pltpu.VMEM((b_per_w, D), jnp.float32),
            pltpu.SemaphoreType.DMA,
            pltpu.SemaphoreType.DMA,
        ],
        compiler_params=pltpu.CompilerParams(needs_layout_passes=False),
    )
    def gather_kernel(idx_hbm, off_hbm, table_hbm, out_hbm,
                      idx_v, off_v, rows2, out_v, sem0, sem1):
        wid = lax.axis_index("s") * _NC + lax.axis_index("c")
        pltpu.sync_copy(idx_hbm.at[pl.ds(wid * n_chunks, n_chunks)], idx_v)
        pltpu.sync_copy(off_hbm.at[wid], off_v)
        sems = [sem0, sem1]
        lanes = lax.iota(jnp.int32, _L)

        def start(j):
            return pltpu.async_copy(
                table_hbm.at[idx_v.at[j]], rows2.at[j % 2], sems[j % 2])

        copies = [start(0)]
        for j in range(n_chunks):
            if j + 1 < n_chunks:
                copies.append(start(j + 1))
            copies[j].wait()
            chunk = rows2.at[j % 2]

            def extract(g, _):
                row0 = g * _L
                rows16 = row0 + lanes
                offs = off_v[pl.ds(j * _CHUNK + row0, _L)]
                out_rows16 = j * _CHUNK + rows16
                for c in range(D):
                    vals = plsc.load_gather(chunk, [rows16, offs + c])
                    plsc.store_scatter(
                        out_v, [out_rows16, jnp.full((_L,), c, jnp.int32)], vals)
                return 0

            lax.fori_loop(0, n_groups, extract, 0)
        pltpu.sync_copy(out_v, out_hbm.at[pl.ds(wid * b_per_w, b_per_w)])

    return gather_kernel(super_idx, sub_off, table128)
